# baseline (device time: 58998 ns/iter reference)
import functools
import os

import jax
import jax.numpy as jnp
from jax import lax
from jax.experimental import pallas as pl
from jax.experimental.pallas import tpu as pltpu

N_DEV = 8
SQ = 1024
SKV = 1024
HQ = 8
DH = 128
CHUNK = SQ // N_DEV
NRES = 4
GRP = SQ // NRES
SCALE = 0.08838834764831843

_SKIP_RS = os.environ.get("K_SKIP_RS") == "1"
_SKIP_AG = os.environ.get("K_SKIP_AG") == "1"


def kernel(x, Wq, K_ext, V_ext, Wo):
    def body(x_ref, wq_ref, k_ref, v_ref, wo_ref, out_ref,
             acc_ref, lw_ref, acc_i8, rs_acc_slots, rs_lw_slots,
             ag_send, ag_slots, ag_sc_send, ag_sc_slots,
             acc_ssem, acc_rsem, l_ssem, l_rsem, ag_ssem, ag_rsem,
             sc_ssem, sc_rsem):
        my = lax.axis_index("i")

        if not (_SKIP_RS and _SKIP_AG):
            barrier = pltpu.get_barrier_semaphore()
            for o in range(1, N_DEV):
                peer = lax.rem(my + o, N_DEV)
                pl.semaphore_signal(barrier, inc=1, device_id=(peer,),
                                    device_id_type=pl.DeviceIdType.MESH)
            pl.semaphore_wait(barrier, N_DEV - 1)

        xb = x_ref[0].astype(jnp.bfloat16)
        wqb = wq_ref[...].astype(jnp.bfloat16)
        q = lax.dot_general(xb, wqb, (((1,), (0,)), ((), ())),
                            preferred_element_type=jnp.float32)
        q = (q * SCALE).astype(jnp.bfloat16)

        qp = q.reshape(NRES, NRES, 64, HQ * DH).transpose(1, 0, 2, 3)
        qp = qp.reshape(NRES, GRP, HQ * DH)
        kp = k_ref[0].astype(jnp.bfloat16).reshape(
            NRES, NRES, 64, HQ, DH).transpose(1, 0, 2, 3, 4)
        kp = kp.reshape(NRES, GRP, HQ, DH)
        vp = v_ref[0].astype(jnp.bfloat16).reshape(
            NRES, NRES, 64, HQ, DH).transpose(1, 0, 2, 3, 4)
        vp = vp.reshape(NRES, GRP, HQ, DH)

        def rs_descriptor(c, hp):
            j = lax.rem(c - my - 1 + N_DEV, N_DEV)
            return pltpu.make_async_remote_copy(
                src_ref=acc_i8.at[2 * hp:2 * hp + 2, pl.ds(c * CHUNK, CHUNK), :],
                dst_ref=rs_acc_slots.at[hp, j],
                send_sem=acc_ssem.at[hp, j], recv_sem=acc_rsem.at[hp, j],
                device_id=(c,), device_id_type=pl.DeviceIdType.MESH)

        def rs_lw_descriptor(c, hp):
            j = lax.rem(c - my - 1 + N_DEV, N_DEV)
            return pltpu.make_async_remote_copy(
                src_ref=lw_ref.at[hp, pl.ds(c * CHUNK, CHUNK), :],
                dst_ref=rs_lw_slots.at[hp, j],
                send_sem=l_ssem.at[hp, j], recv_sem=l_rsem.at[hp, j],
                device_id=(c,), device_id_type=pl.DeviceIdType.MESH)

        myl = pl.ds(my * CHUNK, CHUNK)
        tot_acc = [None] * HQ
        tot_lh = [None] * HQ

        def combine_pair(hp):
            l_loc = lw_ref[hp, myl, :]
            tl = [l_loc[:, 0], l_loc[:, 1]]
            ta = [acc_ref[2 * hp, myl, :], acc_ref[2 * hp + 1, myl, :]]
            if not _SKIP_RS:
                for j in range(N_DEV - 1):
                    pltpu.make_async_remote_copy(
                        src_ref=rs_acc_slots.at[hp, j],
                        dst_ref=rs_acc_slots.at[hp, j],
                        send_sem=acc_ssem.at[hp, j],
                        recv_sem=acc_rsem.at[hp, j], device_id=(my,),
                        device_id_type=pl.DeviceIdType.MESH).wait_recv()
                    pltpu.make_async_remote_copy(
                        src_ref=rs_lw_slots.at[hp, j],
                        dst_ref=rs_lw_slots.at[hp, j],
                        send_sem=l_ssem.at[hp, j],
                        recv_sem=l_rsem.at[hp, j], device_id=(my,),
                        device_id_type=pl.DeviceIdType.MESH).wait_recv()
                for j in range(N_DEV - 1):
                    lwj = rs_lw_slots[hp, j]
                    for k in range(2):
                        ta[k] = ta[k] + (
                            rs_acc_slots[hp, j, k].astype(jnp.float32)
                            * lwj[:, 2 + k, None])
                        tl[k] = tl[k] + lwj[:, k]
            for k in range(2):
                tot_acc[2 * hp + k] = ta[k]
                tot_lh[2 * hp + k] = tl[k]

        for h in range(HQ):
            hp, k = h // 2, h % 2
            for r in range(NRES):
                rrows = pl.ds(r * GRP, GRP)
                qrh = qp[r, :, h * DH:(h + 1) * DH]
                s = lax.dot_general(qrh, kp[r, :, h, :],
                                    (((1,), (1,)), ((), ())),
                                    preferred_element_type=jnp.float32)
                w = jnp.exp(s)
                lw_ref[hp, rrows, k] = jnp.sum(w, axis=1)
                acc_ref[h, rrows, :] = lax.dot_general(
                    w.astype(jnp.bfloat16), vp[r, :, h, :],
                    (((1,), (0,)), ((), ())),
                    preferred_element_type=jnp.float32)
            acc_h = acc_ref[h]
            qsc = (jnp.max(jnp.abs(acc_h), axis=1, keepdims=True)
                   * (1.0 / 127.0) + 1e-20)
            acc_i8[h] = jnp.round(acc_h / qsc).astype(jnp.int8)
            lw_ref[hp, :, 2 + k] = qsc[:, 0]
            if k == 1:
                if not _SKIP_RS:
                    for c in range(N_DEV):
                        @pl.when(my != c)
                        def _(c=c, hp=hp):
                            rs_descriptor(c, hp).start()
                            rs_lw_descriptor(c, hp).start()
                if hp >= 1:
                    combine_pair(hp - 1)

        combine_pair(HQ // 2 - 1)

        ctx_parts = []
        for h in range(HQ):
            ctx_parts.append((tot_acc[h] / tot_lh[h][:, None])
                             .astype(jnp.bfloat16))
        ctxb = jnp.concatenate(ctx_parts, axis=1)

        wob = wo_ref[...].astype(jnp.bfloat16)
        out_chunk = lax.dot_general(ctxb, wob, (((1,), (0,)), ((), ())),
                                    preferred_element_type=jnp.float32)
        row_scale = (jnp.max(jnp.abs(out_chunk), axis=1, keepdims=True)
                     * (1.0 / 127.0) + 1e-20)
        ag_send[...] = jnp.round(out_chunk / row_scale).astype(jnp.int8)
        ag_sc_send[...] = row_scale

        def store_chunk(c, chunk_f32):
            b0 = (8 * lax.rem(c, 2) + lax.div(c, 2)) * 64
            out_ref[0, pl.ds(b0, 64), :] = chunk_f32[:64]
            out_ref[0, pl.ds(b0 + 256, 64), :] = chunk_f32[64:]

        store_chunk(my, out_chunk)

        ag_rdmas = []
        if not _SKIP_AG:
            for o in range(1, N_DEV):
                peer = lax.rem(my + o, N_DEV)
                j = N_DEV - 1 - o
                rdma = pltpu.make_async_remote_copy(
                    src_ref=ag_send, dst_ref=ag_slots.at[j],
                    send_sem=ag_ssem.at[j], recv_sem=ag_rsem.at[j],
                    device_id=(peer,), device_id_type=pl.DeviceIdType.MESH)
                rdma.start()
                rdma_sc = pltpu.make_async_remote_copy(
                    src_ref=ag_sc_send, dst_ref=ag_sc_slots.at[j],
                    send_sem=sc_ssem.at[j], recv_sem=sc_rsem.at[j],
                    device_id=(peer,), device_id_type=pl.DeviceIdType.MESH)
                rdma_sc.start()
                ag_rdmas.extend((rdma, rdma_sc))

            for j in range(N_DEV - 1):
                pltpu.make_async_remote_copy(
                    src_ref=ag_slots.at[j], dst_ref=ag_slots.at[j],
                    send_sem=ag_ssem.at[j], recv_sem=ag_rsem.at[j],
                    device_id=(my,),
                    device_id_type=pl.DeviceIdType.MESH).wait_recv()
                pltpu.make_async_remote_copy(
                    src_ref=ag_sc_slots.at[j], dst_ref=ag_sc_slots.at[j],
                    send_sem=sc_ssem.at[j], recv_sem=sc_rsem.at[j],
                    device_id=(my,),
                    device_id_type=pl.DeviceIdType.MESH).wait_recv()
                src = lax.rem(my + 1 + j, N_DEV)
                store_chunk(src, ag_slots[j].astype(jnp.float32)
                            * ag_sc_slots[j])

        if not _SKIP_RS:
            for c in range(N_DEV):
                @pl.when(my != c)
                def _(c=c):
                    for hp in range(HQ // 2):
                        rs_descriptor(c, hp).wait_send()
                        rs_lw_descriptor(c, hp).wait_send()
        for rdma in ag_rdmas:
            rdma.wait_send()

        if not (_SKIP_RS and _SKIP_AG):
            @functools.partial(pl.run_scoped,
                               second_barrier=pltpu.SemaphoreType.REGULAR)
            def _(second_barrier):
                for o in range(1, N_DEV):
                    peer = lax.rem(my + o, N_DEV)
                    pl.semaphore_signal(second_barrier, inc=1,
                                        device_id=(peer,),
                                        device_id_type=pl.DeviceIdType.MESH)
                pl.semaphore_wait(second_barrier, N_DEV - 1)

    return pl.pallas_call(
        body,
        out_shape=jax.ShapeDtypeStruct((1, SQ, HQ * DH), jnp.float32),
        in_specs=[pl.BlockSpec(memory_space=pltpu.VMEM)] * 5,
        out_specs=pl.BlockSpec(memory_space=pltpu.VMEM),
        scratch_shapes=[
            pltpu.VMEM((HQ, SQ, DH), jnp.float32),
            pltpu.VMEM((HQ // 2, SQ, 4), jnp.float32),
            pltpu.VMEM((HQ, SQ, DH), jnp.int8),
            pltpu.VMEM((HQ // 2, N_DEV - 1, 2, CHUNK, DH),
                       jnp.int8),
            pltpu.VMEM((HQ // 2, N_DEV - 1, CHUNK, 4),
                       jnp.float32),
            pltpu.VMEM((CHUNK, HQ * DH), jnp.int8),
            pltpu.VMEM((N_DEV - 1, CHUNK, HQ * DH), jnp.int8),
            pltpu.VMEM((CHUNK, 1), jnp.float32),
            pltpu.VMEM((N_DEV - 1, CHUNK, 1), jnp.float32),
            pltpu.SemaphoreType.DMA((HQ // 2, N_DEV - 1)),
            pltpu.SemaphoreType.DMA((HQ // 2, N_DEV - 1)),
            pltpu.SemaphoreType.DMA((HQ // 2, N_DEV - 1)),
            pltpu.SemaphoreType.DMA((HQ // 2, N_DEV - 1)),
            pltpu.SemaphoreType.DMA((N_DEV - 1,)),
            pltpu.SemaphoreType.DMA((N_DEV - 1,)),
            pltpu.SemaphoreType.DMA((N_DEV - 1,)),
            pltpu.SemaphoreType.DMA((N_DEV - 1,)),
        ],
        compiler_params=(None if (_SKIP_RS and _SKIP_AG)
                         else pltpu.CompilerParams(collective_id=0)),
    )(x, Wq, K_ext, V_ext, Wo)


# device time: 58813 ns/iter; 1.0031x vs baseline; 1.0031x over previous
import functools
import os

import jax
import jax.numpy as jnp
from jax import lax
from jax.experimental import pallas as pl
from jax.experimental.pallas import tpu as pltpu

N_DEV = 8
SQ = 1024
SKV = 1024
HQ = 8
DH = 128
CHUNK = SQ // N_DEV
NRES = 4
GRP = SQ // NRES
SCALE = 0.08838834764831843

_SKIP_RS = os.environ.get("K_SKIP_RS") == "1"
_SKIP_AG = os.environ.get("K_SKIP_AG") == "1"


def kernel(x, Wq, K_ext, V_ext, Wo):
    def body(x_ref, wq_ref, k_ref, v_ref, wo_ref, out_ref,
             acc_ref, lw_ref, acc_i8, rs_acc_slots, rs_lw_slots,
             ag_send, ag_slots, ag_sc_send, ag_sc_slots,
             acc_ssem, acc_rsem, l_ssem, l_rsem, ag_ssem, ag_rsem,
             sc_ssem, sc_rsem):
        my = lax.axis_index("i")

        if not (_SKIP_RS and _SKIP_AG):
            barrier = pltpu.get_barrier_semaphore()
            for o in range(1, N_DEV):
                peer = lax.rem(my + o, N_DEV)
                pl.semaphore_signal(barrier, inc=1, device_id=(peer,),
                                    device_id_type=pl.DeviceIdType.MESH)
            pl.semaphore_wait(barrier, N_DEV - 1)

        xb = x_ref[0].astype(jnp.bfloat16)
        wqb = wq_ref[...].astype(jnp.bfloat16)
        q = lax.dot_general(xb, wqb, (((1,), (0,)), ((), ())),
                            preferred_element_type=jnp.float32)
        q = (q * SCALE).astype(jnp.bfloat16)

        qp = q.reshape(NRES, NRES, 64, HQ * DH).transpose(1, 0, 2, 3)
        qp = qp.reshape(NRES, GRP, HQ * DH)
        kp = k_ref[0].astype(jnp.bfloat16).reshape(
            NRES, NRES, 64, HQ, DH).transpose(1, 0, 2, 3, 4)
        kp = kp.reshape(NRES, GRP, HQ, DH)
        vp = v_ref[0].astype(jnp.bfloat16).reshape(
            NRES, NRES, 64, HQ, DH).transpose(1, 0, 2, 3, 4)
        vp = vp.reshape(NRES, GRP, HQ, DH)

        def rs_descriptor(c, hp):
            j = lax.rem(c - my - 1 + N_DEV, N_DEV)
            return pltpu.make_async_remote_copy(
                src_ref=acc_i8.at[2 * hp:2 * hp + 2, pl.ds(c * CHUNK, CHUNK), :],
                dst_ref=rs_acc_slots.at[hp, j],
                send_sem=acc_ssem.at[hp, j], recv_sem=acc_rsem.at[hp, j],
                device_id=(c,), device_id_type=pl.DeviceIdType.MESH)

        def rs_lw_descriptor(c, hp):
            j = lax.rem(c - my - 1 + N_DEV, N_DEV)
            return pltpu.make_async_remote_copy(
                src_ref=lw_ref.at[hp, pl.ds(c * CHUNK, CHUNK), :],
                dst_ref=rs_lw_slots.at[hp, j],
                send_sem=l_ssem.at[hp, j], recv_sem=l_rsem.at[hp, j],
                device_id=(c,), device_id_type=pl.DeviceIdType.MESH)

        myl = pl.ds(my * CHUNK, CHUNK)
        tot_acc = [None] * HQ
        tot_lh = [None] * HQ

        def combine_pair(hp):
            l_loc = lw_ref[hp, myl, :]
            tl = [l_loc[:, 0], l_loc[:, 1]]
            ta = [acc_ref[2 * hp, myl, :], acc_ref[2 * hp + 1, myl, :]]
            if not _SKIP_RS:
                for j in range(N_DEV - 1):
                    pltpu.make_async_remote_copy(
                        src_ref=rs_acc_slots.at[hp, j],
                        dst_ref=rs_acc_slots.at[hp, j],
                        send_sem=acc_ssem.at[hp, j],
                        recv_sem=acc_rsem.at[hp, j], device_id=(my,),
                        device_id_type=pl.DeviceIdType.MESH).wait_recv()
                    pltpu.make_async_remote_copy(
                        src_ref=rs_lw_slots.at[hp, j],
                        dst_ref=rs_lw_slots.at[hp, j],
                        send_sem=l_ssem.at[hp, j],
                        recv_sem=l_rsem.at[hp, j], device_id=(my,),
                        device_id_type=pl.DeviceIdType.MESH).wait_recv()
                for j in range(N_DEV - 1):
                    lwj = rs_lw_slots[hp, j]
                    for k in range(2):
                        ta[k] = ta[k] + (
                            rs_acc_slots[hp, j, k].astype(jnp.float32)
                            * lwj[:, 2 + k, None])
                        tl[k] = tl[k] + lwj[:, k]
            for k in range(2):
                tot_acc[2 * hp + k] = ta[k]
                tot_lh[2 * hp + k] = tl[k]

        for h in range(HQ):
            hp, k = h // 2, h % 2
            for r in range(NRES):
                rrows = pl.ds(r * GRP, GRP)
                qrh = qp[r, :, h * DH:(h + 1) * DH]
                s = lax.dot_general(qrh, kp[r, :, h, :],
                                    (((1,), (1,)), ((), ())),
                                    preferred_element_type=jnp.float32)
                w = jnp.exp(s)
                lw_ref[hp, rrows, k] = jnp.sum(w, axis=1)
                acc_ref[h, rrows, :] = lax.dot_general(
                    w.astype(jnp.bfloat16), vp[r, :, h, :],
                    (((1,), (0,)), ((), ())),
                    preferred_element_type=jnp.float32)
            acc_h = acc_ref[h]
            qsc = (jnp.max(jnp.abs(acc_h), axis=1, keepdims=True)
                   * (1.0 / 127.0) + 1e-20)
            acc_i8[h] = jnp.round(acc_h / qsc).astype(jnp.int8)
            lw_ref[hp, :, 2 + k] = qsc[:, 0]
            if k == 1 and not _SKIP_RS:
                for c in range(N_DEV):
                    @pl.when(my != c)
                    def _(c=c, hp=hp):
                        rs_descriptor(c, hp).start()
                        rs_lw_descriptor(c, hp).start()

        for hp in range(HQ // 2):
            combine_pair(hp)

        ctx_parts = []
        for h in range(HQ):
            ctx_parts.append((tot_acc[h] / tot_lh[h][:, None])
                             .astype(jnp.bfloat16))
        ctxb = jnp.concatenate(ctx_parts, axis=1)

        wob = wo_ref[...].astype(jnp.bfloat16)
        out_chunk = lax.dot_general(ctxb, wob, (((1,), (0,)), ((), ())),
                                    preferred_element_type=jnp.float32)
        row_scale = (jnp.max(jnp.abs(out_chunk), axis=1, keepdims=True)
                     * (1.0 / 127.0) + 1e-20)
        ag_send[...] = jnp.round(out_chunk / row_scale).astype(jnp.int8)
        ag_sc_send[...] = row_scale

        def store_chunk(c, chunk_f32):
            b0 = (8 * lax.rem(c, 2) + lax.div(c, 2)) * 64
            out_ref[0, pl.ds(b0, 64), :] = chunk_f32[:64]
            out_ref[0, pl.ds(b0 + 256, 64), :] = chunk_f32[64:]

        store_chunk(my, out_chunk)

        ag_rdmas = []
        if not _SKIP_AG:
            for o in range(1, N_DEV):
                peer = lax.rem(my + o, N_DEV)
                j = N_DEV - 1 - o
                rdma = pltpu.make_async_remote_copy(
                    src_ref=ag_send, dst_ref=ag_slots.at[j],
                    send_sem=ag_ssem.at[j], recv_sem=ag_rsem.at[j],
                    device_id=(peer,), device_id_type=pl.DeviceIdType.MESH)
                rdma.start()
                rdma_sc = pltpu.make_async_remote_copy(
                    src_ref=ag_sc_send, dst_ref=ag_sc_slots.at[j],
                    send_sem=sc_ssem.at[j], recv_sem=sc_rsem.at[j],
                    device_id=(peer,), device_id_type=pl.DeviceIdType.MESH)
                rdma_sc.start()
                ag_rdmas.extend((rdma, rdma_sc))

            for j in range(N_DEV - 1):
                pltpu.make_async_remote_copy(
                    src_ref=ag_slots.at[j], dst_ref=ag_slots.at[j],
                    send_sem=ag_ssem.at[j], recv_sem=ag_rsem.at[j],
                    device_id=(my,),
                    device_id_type=pl.DeviceIdType.MESH).wait_recv()
                pltpu.make_async_remote_copy(
                    src_ref=ag_sc_slots.at[j], dst_ref=ag_sc_slots.at[j],
                    send_sem=sc_ssem.at[j], recv_sem=sc_rsem.at[j],
                    device_id=(my,),
                    device_id_type=pl.DeviceIdType.MESH).wait_recv()
                src = lax.rem(my + 1 + j, N_DEV)
                store_chunk(src, ag_slots[j].astype(jnp.float32)
                            * ag_sc_slots[j])

        if not _SKIP_RS:
            for c in range(N_DEV):
                @pl.when(my != c)
                def _(c=c):
                    for hp in range(HQ // 2):
                        rs_descriptor(c, hp).wait_send()
                        rs_lw_descriptor(c, hp).wait_send()
        for rdma in ag_rdmas:
            rdma.wait_send()

        if not (_SKIP_RS and _SKIP_AG):
            @functools.partial(pl.run_scoped,
                               second_barrier=pltpu.SemaphoreType.REGULAR)
            def _(second_barrier):
                for o in range(1, N_DEV):
                    peer = lax.rem(my + o, N_DEV)
                    pl.semaphore_signal(second_barrier, inc=1,
                                        device_id=(peer,),
                                        device_id_type=pl.DeviceIdType.MESH)
                pl.semaphore_wait(second_barrier, N_DEV - 1)

    return pl.pallas_call(
        body,
        out_shape=jax.ShapeDtypeStruct((1, SQ, HQ * DH), jnp.float32),
        in_specs=[pl.BlockSpec(memory_space=pltpu.VMEM)] * 5,
        out_specs=pl.BlockSpec(memory_space=pltpu.VMEM),
        scratch_shapes=[
            pltpu.VMEM((HQ, SQ, DH), jnp.float32),
            pltpu.VMEM((HQ // 2, SQ, 4), jnp.float32),
            pltpu.VMEM((HQ, SQ, DH), jnp.int8),
            pltpu.VMEM((HQ // 2, N_DEV - 1, 2, CHUNK, DH),
                       jnp.int8),
            pltpu.VMEM((HQ // 2, N_DEV - 1, CHUNK, 4),
                       jnp.float32),
            pltpu.VMEM((CHUNK, HQ * DH), jnp.int8),
            pltpu.VMEM((N_DEV - 1, CHUNK, HQ * DH), jnp.int8),
            pltpu.VMEM((CHUNK, 1), jnp.float32),
            pltpu.VMEM((N_DEV - 1, CHUNK, 1), jnp.float32),
            pltpu.SemaphoreType.DMA((HQ // 2, N_DEV - 1)),
            pltpu.SemaphoreType.DMA((HQ // 2, N_DEV - 1)),
            pltpu.SemaphoreType.DMA((HQ // 2, N_DEV - 1)),
            pltpu.SemaphoreType.DMA((HQ // 2, N_DEV - 1)),
            pltpu.SemaphoreType.DMA((N_DEV - 1,)),
            pltpu.SemaphoreType.DMA((N_DEV - 1,)),
            pltpu.SemaphoreType.DMA((N_DEV - 1,)),
            pltpu.SemaphoreType.DMA((N_DEV - 1,)),
        ],
        compiler_params=(None if (_SKIP_RS and _SKIP_AG)
                         else pltpu.CompilerParams(collective_id=0)),
    )(x, Wq, K_ext, V_ext, Wo)


# device time: 56632 ns/iter; 1.0418x vs baseline; 1.0385x over previous
import functools
import os

import jax
import jax.numpy as jnp
from jax import lax
from jax.experimental import pallas as pl
from jax.experimental.pallas import tpu as pltpu

N_DEV = 8
SQ = 1024
SKV = 1024
HQ = 8
DH = 128
CHUNK = SQ // N_DEV
NRES = 4
GRP = SQ // NRES
SCALE = 0.08838834764831843

_SKIP_RS = os.environ.get("K_SKIP_RS") == "1"
_SKIP_AG = os.environ.get("K_SKIP_AG") == "1"


def kernel(x, Wq, K_ext, V_ext, Wo):
    def body(x_ref, wq_ref, k_ref, v_ref, wo_ref, out_ref,
             acc_ref, lw_ref, acc_i8, rs_acc_slots, rs_lw_slots,
             ag_send, ag_slots, ag_sc_send, ag_sc_slots,
             acc_ssem, acc_rsem, l_ssem, l_rsem, ag_ssem, ag_rsem,
             sc_ssem, sc_rsem):
        my = lax.axis_index("i")

        if not (_SKIP_RS and _SKIP_AG):
            barrier = pltpu.get_barrier_semaphore()
            for o in range(1, N_DEV):
                peer = lax.rem(my + o, N_DEV)
                pl.semaphore_signal(barrier, inc=1, device_id=(peer,),
                                    device_id_type=pl.DeviceIdType.MESH)
            pl.semaphore_wait(barrier, N_DEV - 1)

        xb = x_ref[0].astype(jnp.bfloat16)
        wqb = wq_ref[...].astype(jnp.bfloat16)
        q = lax.dot_general(xb, wqb, (((1,), (0,)), ((), ())),
                            preferred_element_type=jnp.float32)
        q = (q * SCALE).astype(jnp.bfloat16)

        qp = q.reshape(NRES, NRES, 64, HQ * DH).transpose(1, 0, 2, 3)
        qp = qp.reshape(NRES, GRP, HQ * DH)
        kp = k_ref[0].astype(jnp.bfloat16).reshape(
            NRES, NRES, 64, HQ, DH).transpose(1, 0, 2, 3, 4)
        kp = kp.reshape(NRES, GRP, HQ, DH)
        vp = v_ref[0].astype(jnp.bfloat16).reshape(
            NRES, NRES, 64, HQ, DH).transpose(1, 0, 2, 3, 4)
        vp = vp.reshape(NRES, GRP, HQ, DH)

        def rs_descriptor(c, hp):
            j = lax.rem(c - my - 1 + N_DEV, N_DEV)
            return pltpu.make_async_remote_copy(
                src_ref=acc_i8.at[2 * hp:2 * hp + 2, pl.ds(c * CHUNK, CHUNK), :],
                dst_ref=rs_acc_slots.at[hp, j],
                send_sem=acc_ssem.at[hp, j], recv_sem=acc_rsem.at[hp, j],
                device_id=(c,), device_id_type=pl.DeviceIdType.MESH)

        def rs_lw_descriptor(c, hp):
            j = lax.rem(c - my - 1 + N_DEV, N_DEV)
            return pltpu.make_async_remote_copy(
                src_ref=lw_ref.at[hp, :, pl.ds(c * CHUNK, CHUNK)],
                dst_ref=rs_lw_slots.at[hp, j],
                send_sem=l_ssem.at[hp, j], recv_sem=l_rsem.at[hp, j],
                device_id=(c,), device_id_type=pl.DeviceIdType.MESH)

        myl = pl.ds(my * CHUNK, CHUNK)
        tot_acc = [None] * HQ
        tot_lh = [None] * HQ

        def combine_pair(hp):
            l_loc = lw_ref[hp, :, myl]
            tl = [l_loc[0], l_loc[1]]
            ta = [acc_ref[2 * hp, myl, :], acc_ref[2 * hp + 1, myl, :]]
            if not _SKIP_RS:
                for j in range(N_DEV - 1):
                    pltpu.make_async_remote_copy(
                        src_ref=rs_acc_slots.at[hp, j],
                        dst_ref=rs_acc_slots.at[hp, j],
                        send_sem=acc_ssem.at[hp, j],
                        recv_sem=acc_rsem.at[hp, j], device_id=(my,),
                        device_id_type=pl.DeviceIdType.MESH).wait_recv()
                    pltpu.make_async_remote_copy(
                        src_ref=rs_lw_slots.at[hp, j],
                        dst_ref=rs_lw_slots.at[hp, j],
                        send_sem=l_ssem.at[hp, j],
                        recv_sem=l_rsem.at[hp, j], device_id=(my,),
                        device_id_type=pl.DeviceIdType.MESH).wait_recv()
                for j in range(N_DEV - 1):
                    lwj = rs_lw_slots[hp, j]
                    for k in range(2):
                        ta[k] = ta[k] + (
                            rs_acc_slots[hp, j, k].astype(jnp.float32)
                            * lwj[2 + k][:, None])
                        tl[k] = tl[k] + lwj[k]
            for k in range(2):
                tot_acc[2 * hp + k] = ta[k]
                tot_lh[2 * hp + k] = tl[k]

        for h in range(HQ):
            hp, k = h // 2, h % 2
            for r in range(NRES):
                rrows = pl.ds(r * GRP, GRP)
                qrh = qp[r, :, h * DH:(h + 1) * DH]
                s = lax.dot_general(qrh, kp[r, :, h, :],
                                    (((1,), (1,)), ((), ())),
                                    preferred_element_type=jnp.float32)
                w = jnp.exp(s)
                lw_ref[hp, k, rrows] = jnp.sum(w, axis=1)
                acc_ref[h, rrows, :] = lax.dot_general(
                    w.astype(jnp.bfloat16), vp[r, :, h, :],
                    (((1,), (0,)), ((), ())),
                    preferred_element_type=jnp.float32)
            acc_h = acc_ref[h]
            qsc = (jnp.max(jnp.abs(acc_h), axis=1, keepdims=True)
                   * (1.0 / 127.0) + 1e-20)
            acc_i8[h] = jnp.round(acc_h / qsc).astype(jnp.int8)
            lw_ref[hp, 2 + k, :] = qsc[:, 0]
            if k == 1 and not _SKIP_RS:
                for c in range(N_DEV):
                    @pl.when(my != c)
                    def _(c=c, hp=hp):
                        rs_descriptor(c, hp).start()
                        rs_lw_descriptor(c, hp).start()

        for hp in range(HQ // 2):
            combine_pair(hp)

        ctx_parts = []
        for h in range(HQ):
            ctx_parts.append((tot_acc[h] / tot_lh[h][:, None])
                             .astype(jnp.bfloat16))
        ctxb = jnp.concatenate(ctx_parts, axis=1)

        wob = wo_ref[...].astype(jnp.bfloat16)
        out_chunk = lax.dot_general(ctxb, wob, (((1,), (0,)), ((), ())),
                                    preferred_element_type=jnp.float32)
        row_scale = (jnp.max(jnp.abs(out_chunk), axis=1, keepdims=True)
                     * (1.0 / 127.0) + 1e-20)
        ag_send[...] = jnp.round(out_chunk / row_scale).astype(jnp.int8)
        ag_sc_send[...] = row_scale[:, 0]

        def store_chunk(c, chunk_f32):
            b0 = (8 * lax.rem(c, 2) + lax.div(c, 2)) * 64
            out_ref[0, pl.ds(b0, 64), :] = chunk_f32[:64]
            out_ref[0, pl.ds(b0 + 256, 64), :] = chunk_f32[64:]

        store_chunk(my, out_chunk)

        ag_rdmas = []
        if not _SKIP_AG:
            for o in range(1, N_DEV):
                peer = lax.rem(my + o, N_DEV)
                j = N_DEV - 1 - o
                rdma = pltpu.make_async_remote_copy(
                    src_ref=ag_send, dst_ref=ag_slots.at[j],
                    send_sem=ag_ssem.at[j], recv_sem=ag_rsem.at[j],
                    device_id=(peer,), device_id_type=pl.DeviceIdType.MESH)
                rdma.start()
                rdma_sc = pltpu.make_async_remote_copy(
                    src_ref=ag_sc_send, dst_ref=ag_sc_slots.at[j],
                    send_sem=sc_ssem.at[j], recv_sem=sc_rsem.at[j],
                    device_id=(peer,), device_id_type=pl.DeviceIdType.MESH)
                rdma_sc.start()
                ag_rdmas.extend((rdma, rdma_sc))

            for j in range(N_DEV - 1):
                pltpu.make_async_remote_copy(
                    src_ref=ag_slots.at[j], dst_ref=ag_slots.at[j],
                    send_sem=ag_ssem.at[j], recv_sem=ag_rsem.at[j],
                    device_id=(my,),
                    device_id_type=pl.DeviceIdType.MESH).wait_recv()
                pltpu.make_async_remote_copy(
                    src_ref=ag_sc_slots.at[j], dst_ref=ag_sc_slots.at[j],
                    send_sem=sc_ssem.at[j], recv_sem=sc_rsem.at[j],
                    device_id=(my,),
                    device_id_type=pl.DeviceIdType.MESH).wait_recv()
                src = lax.rem(my + 1 + j, N_DEV)
                store_chunk(src, ag_slots[j].astype(jnp.float32)
                            * ag_sc_slots[j][:, None])

        if not _SKIP_RS:
            for c in range(N_DEV):
                @pl.when(my != c)
                def _(c=c):
                    for hp in range(HQ // 2):
                        rs_descriptor(c, hp).wait_send()
                        rs_lw_descriptor(c, hp).wait_send()
        for rdma in ag_rdmas:
            rdma.wait_send()

        if not (_SKIP_RS and _SKIP_AG):
            @functools.partial(pl.run_scoped,
                               second_barrier=pltpu.SemaphoreType.REGULAR)
            def _(second_barrier):
                for o in range(1, N_DEV):
                    peer = lax.rem(my + o, N_DEV)
                    pl.semaphore_signal(second_barrier, inc=1,
                                        device_id=(peer,),
                                        device_id_type=pl.DeviceIdType.MESH)
                pl.semaphore_wait(second_barrier, N_DEV - 1)

    return pl.pallas_call(
        body,
        out_shape=jax.ShapeDtypeStruct((1, SQ, HQ * DH), jnp.float32),
        in_specs=[pl.BlockSpec(memory_space=pltpu.VMEM)] * 5,
        out_specs=pl.BlockSpec(memory_space=pltpu.VMEM),
        scratch_shapes=[
            pltpu.VMEM((HQ, SQ, DH), jnp.float32),
            pltpu.VMEM((HQ // 2, 4, SQ), jnp.float32),
            pltpu.VMEM((HQ, SQ, DH), jnp.int8),
            pltpu.VMEM((HQ // 2, N_DEV - 1, 2, CHUNK, DH),
                       jnp.int8),
            pltpu.VMEM((HQ // 2, N_DEV - 1, 4, CHUNK),
                       jnp.float32),
            pltpu.VMEM((CHUNK, HQ * DH), jnp.int8),
            pltpu.VMEM((N_DEV - 1, CHUNK, HQ * DH), jnp.int8),
            pltpu.VMEM((CHUNK,), jnp.float32),
            pltpu.VMEM((N_DEV - 1, CHUNK), jnp.float32),
            pltpu.SemaphoreType.DMA((HQ // 2, N_DEV - 1)),
            pltpu.SemaphoreType.DMA((HQ // 2, N_DEV - 1)),
            pltpu.SemaphoreType.DMA((HQ // 2, N_DEV - 1)),
            pltpu.SemaphoreType.DMA((HQ // 2, N_DEV - 1)),
            pltpu.SemaphoreType.DMA((N_DEV - 1,)),
            pltpu.SemaphoreType.DMA((N_DEV - 1,)),
            pltpu.SemaphoreType.DMA((N_DEV - 1,)),
            pltpu.SemaphoreType.DMA((N_DEV - 1,)),
        ],
        compiler_params=(None if (_SKIP_RS and _SKIP_AG)
                         else pltpu.CompilerParams(collective_id=0)),
    )(x, Wq, K_ext, V_ext, Wo)


# device time: 48217 ns/iter; 1.2236x vs baseline; 1.1745x over previous
import functools
import os

import jax
import jax.numpy as jnp
from jax import lax
from jax.experimental import pallas as pl
from jax.experimental.pallas import tpu as pltpu

N_DEV = 8
SQ = 1024
SKV = 1024
HQ = 8
DH = 128
CHUNK = SQ // N_DEV
NRES = 4
GRP = SQ // NRES
SCALE = 0.08838834764831843

_SKIP_RS = os.environ.get("K_SKIP_RS") == "1"
_SKIP_AG = os.environ.get("K_SKIP_AG") == "1"


def kernel(x, Wq, K_ext, V_ext, Wo):
    def body(x_ref, wq_ref, k_ref, v_ref, wo_ref, out_ref,
             acc_ref, lw_ref, acc_i8, rs_acc_slots, rs_lw_slots,
             ag_send, ag_slots, ag_sc_send, ag_sc_slots,
             acc_ssem, acc_rsem, l_ssem, l_rsem, ag_ssem, ag_rsem,
             sc_ssem, sc_rsem):
        my = lax.axis_index("i")

        if not (_SKIP_RS and _SKIP_AG):
            barrier = pltpu.get_barrier_semaphore()
            for o in range(1, N_DEV):
                peer = lax.rem(my + o, N_DEV)
                pl.semaphore_signal(barrier, inc=1, device_id=(peer,),
                                    device_id_type=pl.DeviceIdType.MESH)
            pl.semaphore_wait(barrier, N_DEV - 1)

        xb = x_ref[0].astype(jnp.bfloat16)
        wqb = wq_ref[...].astype(jnp.bfloat16)
        q = lax.dot_general(xb, wqb, (((1,), (0,)), ((), ())),
                            preferred_element_type=jnp.float32)
        q = (q * SCALE).astype(jnp.bfloat16)

        qp = q.reshape(NRES, NRES, 64, HQ * DH).transpose(1, 0, 2, 3)
        qp = qp.reshape(NRES, GRP, HQ * DH)
        kp = k_ref[0].astype(jnp.bfloat16).reshape(
            NRES, NRES, 64, HQ, DH).transpose(1, 0, 2, 3, 4)
        kp = kp.reshape(NRES, GRP, HQ, DH)
        vp = v_ref[0].astype(jnp.bfloat16).reshape(
            NRES, NRES, 64, HQ, DH).transpose(1, 0, 2, 3, 4)
        vp = vp.reshape(NRES, GRP, HQ, DH)

        def rs_descriptor(c, hp):
            j = lax.rem(c - my - 1 + N_DEV, N_DEV)
            return pltpu.make_async_remote_copy(
                src_ref=acc_i8.at[2 * hp:2 * hp + 2, pl.ds(c * CHUNK, CHUNK), :],
                dst_ref=rs_acc_slots.at[hp, j],
                send_sem=acc_ssem.at[hp, j], recv_sem=acc_rsem.at[hp, j],
                device_id=(c,), device_id_type=pl.DeviceIdType.MESH)

        def rs_lw_descriptor(c):
            j = lax.rem(c - my - 1 + N_DEV, N_DEV)
            return pltpu.make_async_remote_copy(
                src_ref=lw_ref.at[pl.ds(c * CHUNK, CHUNK)],
                dst_ref=rs_lw_slots.at[j],
                send_sem=l_ssem.at[j], recv_sem=l_rsem.at[j],
                device_id=(c,), device_id_type=pl.DeviceIdType.MESH)

        for h in range(HQ):
            for r in range(NRES):
                rrows = pl.ds(r * GRP, GRP)
                qrh = qp[r, :, h * DH:(h + 1) * DH]
                s = lax.dot_general(qrh, kp[r, :, h, :],
                                    (((1,), (1,)), ((), ())),
                                    preferred_element_type=jnp.float32)
                w = jnp.exp(s)
                lw_ref[rrows, h] = jnp.sum(w, axis=1)
                acc_ref[h, rrows, :] = lax.dot_general(
                    w.astype(jnp.bfloat16), vp[r, :, h, :],
                    (((1,), (0,)), ((), ())),
                    preferred_element_type=jnp.float32)
            acc_h = acc_ref[h]
            qsc = (jnp.max(jnp.abs(acc_h), axis=1, keepdims=True)
                   * (1.0 / 127.0) + 1e-20)
            acc_i8[h] = jnp.round(acc_h / qsc).astype(jnp.int8)
            lw_ref[:, 8 + h] = qsc[:, 0]
            if not _SKIP_RS and h % 2 == 1:
                for c in range(N_DEV):
                    @pl.when(my != c)
                    def _(c=c, hp=h // 2):
                        rs_descriptor(c, hp).start()

        if not _SKIP_RS:
            for c in range(N_DEV):
                @pl.when(my != c)
                def _(c=c):
                    rs_lw_descriptor(c).start()

        myl = pl.ds(my * CHUNK, CHUNK)
        if not _SKIP_RS:
            for hp in range(HQ // 2):
                for j in range(N_DEV - 1):
                    pltpu.make_async_remote_copy(
                        src_ref=rs_acc_slots.at[hp, j],
                        dst_ref=rs_acc_slots.at[hp, j],
                        send_sem=acc_ssem.at[hp, j], recv_sem=acc_rsem.at[hp, j],
                        device_id=(my,),
                        device_id_type=pl.DeviceIdType.MESH).wait_recv()
            for j in range(N_DEV - 1):
                pltpu.make_async_remote_copy(
                    src_ref=rs_lw_slots.at[j], dst_ref=rs_lw_slots.at[j],
                    send_sem=l_ssem.at[j], recv_sem=l_rsem.at[j],
                    device_id=(my,),
                    device_id_type=pl.DeviceIdType.MESH).wait_recv()

        tot_l = lw_ref[myl, 0:8]
        lw_in = []
        if not _SKIP_RS:
            lw_in = [rs_lw_slots[j] for j in range(N_DEV - 1)]
            for j in range(N_DEV - 1):
                tot_l = tot_l + lw_in[j][:, 0:8]

        ctx_parts = []
        for h in range(HQ):
            tot_h = acc_ref[h, myl, :]
            if not _SKIP_RS:
                for j in range(N_DEV - 1):
                    tot_h = tot_h + (
                        rs_acc_slots[h // 2, j, h % 2].astype(jnp.float32)
                        * lw_in[j][:, 8 + h, None])
            ctx_parts.append((tot_h / tot_l[:, h, None]).astype(jnp.bfloat16))
        ctxb = jnp.concatenate(ctx_parts, axis=1)

        wob = wo_ref[...].astype(jnp.bfloat16)
        out_chunk = lax.dot_general(ctxb, wob, (((1,), (0,)), ((), ())),
                                    preferred_element_type=jnp.float32)
        row_scale = (jnp.max(jnp.abs(out_chunk), axis=1, keepdims=True)
                     * (1.0 / 127.0) + 1e-20)
        ag_send[...] = jnp.round(out_chunk / row_scale).astype(jnp.int8)
        ag_sc_send[...] = row_scale[:, 0]

        def store_chunk(c, chunk_f32):
            b0 = (8 * lax.rem(c, 2) + lax.div(c, 2)) * 64
            out_ref[0, pl.ds(b0, 64), :] = chunk_f32[:64]
            out_ref[0, pl.ds(b0 + 256, 64), :] = chunk_f32[64:]

        store_chunk(my, out_chunk)

        ag_rdmas = []
        if not _SKIP_AG:
            for o in range(1, N_DEV):
                peer = lax.rem(my + o, N_DEV)
                j = N_DEV - 1 - o
                rdma = pltpu.make_async_remote_copy(
                    src_ref=ag_send, dst_ref=ag_slots.at[j],
                    send_sem=ag_ssem.at[j], recv_sem=ag_rsem.at[j],
                    device_id=(peer,), device_id_type=pl.DeviceIdType.MESH)
                rdma.start()
                rdma_sc = pltpu.make_async_remote_copy(
                    src_ref=ag_sc_send, dst_ref=ag_sc_slots.at[j],
                    send_sem=sc_ssem.at[j], recv_sem=sc_rsem.at[j],
                    device_id=(peer,), device_id_type=pl.DeviceIdType.MESH)
                rdma_sc.start()
                ag_rdmas.extend((rdma, rdma_sc))

            for j in range(N_DEV - 1):
                pltpu.make_async_remote_copy(
                    src_ref=ag_slots.at[j], dst_ref=ag_slots.at[j],
                    send_sem=ag_ssem.at[j], recv_sem=ag_rsem.at[j],
                    device_id=(my,),
                    device_id_type=pl.DeviceIdType.MESH).wait_recv()
                pltpu.make_async_remote_copy(
                    src_ref=ag_sc_slots.at[j], dst_ref=ag_sc_slots.at[j],
                    send_sem=sc_ssem.at[j], recv_sem=sc_rsem.at[j],
                    device_id=(my,),
                    device_id_type=pl.DeviceIdType.MESH).wait_recv()
                src = lax.rem(my + 1 + j, N_DEV)
                store_chunk(src, ag_slots[j].astype(jnp.float32)
                            * ag_sc_slots[j][:, None])

        if not _SKIP_RS:
            for c in range(N_DEV):
                @pl.when(my != c)
                def _(c=c):
                    for hp in range(HQ // 2):
                        rs_descriptor(c, hp).wait_send()
                    rs_lw_descriptor(c).wait_send()
        for rdma in ag_rdmas:
            rdma.wait_send()

        if not (_SKIP_RS and _SKIP_AG):
            @functools.partial(pl.run_scoped,
                               second_barrier=pltpu.SemaphoreType.REGULAR)
            def _(second_barrier):
                for o in range(1, N_DEV):
                    peer = lax.rem(my + o, N_DEV)
                    pl.semaphore_signal(second_barrier, inc=1,
                                        device_id=(peer,),
                                        device_id_type=pl.DeviceIdType.MESH)
                pl.semaphore_wait(second_barrier, N_DEV - 1)

    return pl.pallas_call(
        body,
        out_shape=jax.ShapeDtypeStruct((1, SQ, HQ * DH), jnp.float32),
        in_specs=[pl.BlockSpec(memory_space=pltpu.VMEM)] * 5,
        out_specs=pl.BlockSpec(memory_space=pltpu.VMEM),
        scratch_shapes=[
            pltpu.VMEM((HQ, SQ, DH), jnp.float32),
            pltpu.VMEM((SQ, 2 * HQ), jnp.float32),
            pltpu.VMEM((HQ, SQ, DH), jnp.int8),
            pltpu.VMEM((HQ // 2, N_DEV - 1, 2, CHUNK, DH),
                       jnp.int8),
            pltpu.VMEM((N_DEV - 1, CHUNK, 2 * HQ), jnp.float32),
            pltpu.VMEM((CHUNK, HQ * DH), jnp.int8),
            pltpu.VMEM((N_DEV - 1, CHUNK, HQ * DH), jnp.int8),
            pltpu.VMEM((CHUNK,), jnp.float32),
            pltpu.VMEM((N_DEV - 1, CHUNK), jnp.float32),
            pltpu.SemaphoreType.DMA((HQ // 2, N_DEV - 1)),
            pltpu.SemaphoreType.DMA((HQ // 2, N_DEV - 1)),
            pltpu.SemaphoreType.DMA((N_DEV - 1,)),
            pltpu.SemaphoreType.DMA((N_DEV - 1,)),
            pltpu.SemaphoreType.DMA((N_DEV - 1,)),
            pltpu.SemaphoreType.DMA((N_DEV - 1,)),
            pltpu.SemaphoreType.DMA((N_DEV - 1,)),
            pltpu.SemaphoreType.DMA((N_DEV - 1,)),
        ],
        compiler_params=(None if (_SKIP_RS and _SKIP_AG)
                         else pltpu.CompilerParams(collective_id=0)),
    )(x, Wq, K_ext, V_ext, Wo)


# device time: 45202 ns/iter; 1.3052x vs baseline; 1.0667x over previous
import functools
import os

import jax
import jax.numpy as jnp
from jax import lax
from jax.experimental import pallas as pl
from jax.experimental.pallas import tpu as pltpu

N_DEV = 8
SQ = 1024
SKV = 1024
HQ = 8
DH = 128
CHUNK = SQ // N_DEV
NRES = 4
GRP = SQ // NRES
SCALE = 0.08838834764831843

_SKIP_RS = os.environ.get("K_SKIP_RS") == "1"
_SKIP_AG = os.environ.get("K_SKIP_AG") == "1"


def kernel(x, Wq, K_ext, V_ext, Wo):
    def body(x_ref, wq_ref, k_ref, v_ref, wo_ref, out_ref,
             acc_ref, lw_ref, acc_i8, rs_acc_slots, rs_lw_slots,
             ag_send, ag_slots, ag_sc_send, ag_sc_slots,
             acc_ssem, acc_rsem, l_ssem, l_rsem, ag_ssem, ag_rsem,
             sc_ssem, sc_rsem):
        my = lax.axis_index("i")

        if not (_SKIP_RS and _SKIP_AG):
            barrier = pltpu.get_barrier_semaphore()
            for o in range(1, N_DEV):
                peer = lax.rem(my + o, N_DEV)
                pl.semaphore_signal(barrier, inc=1, device_id=(peer,),
                                    device_id_type=pl.DeviceIdType.MESH)
            pl.semaphore_wait(barrier, N_DEV - 1)

        xb = x_ref[0].astype(jnp.bfloat16)
        wqb = wq_ref[...].astype(jnp.bfloat16)
        q = lax.dot_general(xb, wqb, (((1,), (0,)), ((), ())),
                            preferred_element_type=jnp.float32)
        q = (q * SCALE).astype(jnp.bfloat16)

        qp = q.reshape(NRES, NRES, 64, HQ * DH).transpose(1, 0, 2, 3)
        qp = qp.reshape(NRES, GRP, HQ * DH)
        kp = k_ref[0].astype(jnp.bfloat16).reshape(
            NRES, NRES, 64, HQ, DH).transpose(1, 0, 2, 3, 4)
        kp = kp.reshape(NRES, GRP, HQ, DH)
        vp = v_ref[0].astype(jnp.bfloat16).reshape(
            NRES, NRES, 64, HQ, DH).transpose(1, 0, 2, 3, 4)
        vp = vp.reshape(NRES, GRP, HQ, DH)

        def rs_descriptor(c, hp):
            j = lax.rem(c - my - 1 + N_DEV, N_DEV)
            return pltpu.make_async_remote_copy(
                src_ref=acc_i8.at[2 * hp:2 * hp + 2, pl.ds(c * CHUNK, CHUNK), :],
                dst_ref=rs_acc_slots.at[hp, j],
                send_sem=acc_ssem.at[hp, j], recv_sem=acc_rsem.at[hp, j],
                device_id=(c,), device_id_type=pl.DeviceIdType.MESH)

        def rs_lw_descriptor(c):
            j = lax.rem(c - my - 1 + N_DEV, N_DEV)
            return pltpu.make_async_remote_copy(
                src_ref=lw_ref.at[pl.ds(c * CHUNK, CHUNK)],
                dst_ref=rs_lw_slots.at[j],
                send_sem=l_ssem.at[j], recv_sem=l_rsem.at[j],
                device_id=(c,), device_id_type=pl.DeviceIdType.MESH)

        for h in range(HQ):
            for r in range(NRES):
                rrows = pl.ds(r * GRP, GRP)
                qrh = qp[r, :, h * DH:(h + 1) * DH]
                s = lax.dot_general(qrh, kp[r, :, h, :],
                                    (((1,), (1,)), ((), ())),
                                    preferred_element_type=jnp.float32)
                w = jnp.exp(s)
                lw_ref[rrows, h] = jnp.sum(w, axis=1)
                acc_ref[h, rrows, :] = lax.dot_general(
                    w.astype(jnp.bfloat16), vp[r, :, h, :],
                    (((1,), (0,)), ((), ())),
                    preferred_element_type=jnp.float32)
            acc_h = acc_ref[h]
            qsc = (jnp.max(jnp.abs(acc_h), axis=1, keepdims=True)
                   * (1.0 / 127.0) + 1e-20)
            acc_i8[h] = jnp.round(acc_h / qsc).astype(jnp.int8)
            lw_ref[:, 8 + h] = qsc[:, 0]
            if not _SKIP_RS and h % 2 == 1:
                for c in range(N_DEV):
                    @pl.when(my != c)
                    def _(c=c, hp=h // 2):
                        rs_descriptor(c, hp).start()

        if not _SKIP_RS:
            for c in range(N_DEV):
                @pl.when(my != c)
                def _(c=c):
                    rs_lw_descriptor(c).start()

        myl = pl.ds(my * CHUNK, CHUNK)
        if not _SKIP_RS:
            for hp in range(HQ // 2):
                for j in range(N_DEV - 1):
                    pltpu.make_async_remote_copy(
                        src_ref=rs_acc_slots.at[hp, j],
                        dst_ref=rs_acc_slots.at[hp, j],
                        send_sem=acc_ssem.at[hp, j], recv_sem=acc_rsem.at[hp, j],
                        device_id=(my,),
                        device_id_type=pl.DeviceIdType.MESH).wait_recv()
            for j in range(N_DEV - 1):
                pltpu.make_async_remote_copy(
                    src_ref=rs_lw_slots.at[j], dst_ref=rs_lw_slots.at[j],
                    send_sem=l_ssem.at[j], recv_sem=l_rsem.at[j],
                    device_id=(my,),
                    device_id_type=pl.DeviceIdType.MESH).wait_recv()

        tot_l = lw_ref[myl, 0:8]
        lw_in = []
        if not _SKIP_RS:
            lw_in = [rs_lw_slots[j] for j in range(N_DEV - 1)]
            for j in range(N_DEV - 1):
                tot_l = tot_l + lw_in[j][:, 0:8]

        ctx_parts = []
        for h in range(HQ):
            tot_h = acc_ref[h, myl, :]
            if not _SKIP_RS:
                for j in range(N_DEV - 1):
                    tot_h = tot_h + (
                        rs_acc_slots[h // 2, j, h % 2].astype(jnp.float32)
                        * lw_in[j][:, 8 + h, None])
            ctx_parts.append((tot_h / tot_l[:, h, None]).astype(jnp.bfloat16))
        ctxb = jnp.concatenate(ctx_parts, axis=1)

        wob = wo_ref[...].astype(jnp.bfloat16)
        out_chunk = lax.dot_general(ctxb, wob, (((1,), (0,)), ((), ())),
                                    preferred_element_type=jnp.float32)
        row_scale = (jnp.max(jnp.abs(out_chunk), axis=1, keepdims=True)
                     * (1.0 / 127.0) + 1e-20)
        ag_send[...] = jnp.round(out_chunk / row_scale).astype(jnp.int8)
        ag_sc_send[...] = row_scale[:, 0]

        def store_chunk(c, chunk_f32):
            b0 = (8 * lax.rem(c, 2) + lax.div(c, 2)) * 64
            out_ref[0, pl.ds(b0, 64), :] = chunk_f32[:64]
            out_ref[0, pl.ds(b0 + 256, 64), :] = chunk_f32[64:]

        store_chunk(my, out_chunk)

        ag_rdmas = []
        if not _SKIP_AG:
            for o in range(1, N_DEV):
                peer = lax.rem(my + o, N_DEV)
                j = N_DEV - 1 - o
                rdma = pltpu.make_async_remote_copy(
                    src_ref=ag_send, dst_ref=ag_slots.at[j],
                    send_sem=ag_ssem.at[j], recv_sem=ag_rsem.at[j],
                    device_id=(peer,), device_id_type=pl.DeviceIdType.MESH)
                rdma.start()
                rdma_sc = pltpu.make_async_remote_copy(
                    src_ref=ag_sc_send, dst_ref=ag_sc_slots.at[j],
                    send_sem=sc_ssem.at[j], recv_sem=sc_rsem.at[j],
                    device_id=(peer,), device_id_type=pl.DeviceIdType.MESH)
                rdma_sc.start()
                ag_rdmas.extend((rdma, rdma_sc))

            for j in range(N_DEV - 1):
                pltpu.make_async_remote_copy(
                    src_ref=ag_slots.at[j], dst_ref=ag_slots.at[j],
                    send_sem=ag_ssem.at[j], recv_sem=ag_rsem.at[j],
                    device_id=(my,),
                    device_id_type=pl.DeviceIdType.MESH).wait_recv()
                pltpu.make_async_remote_copy(
                    src_ref=ag_sc_slots.at[j], dst_ref=ag_sc_slots.at[j],
                    send_sem=sc_ssem.at[j], recv_sem=sc_rsem.at[j],
                    device_id=(my,),
                    device_id_type=pl.DeviceIdType.MESH).wait_recv()
                src = lax.rem(my + 1 + j, N_DEV)
                store_chunk(src, ag_slots[j].astype(jnp.float32)
                            * ag_sc_slots[j][:, None])

        if not _SKIP_RS:
            for c in range(N_DEV):
                @pl.when(my != c)
                def _(c=c):
                    for hp in range(HQ // 2):
                        rs_descriptor(c, hp).wait_send()
                    rs_lw_descriptor(c).wait_send()
        for rdma in ag_rdmas:
            rdma.wait_send()


    return pl.pallas_call(
        body,
        out_shape=jax.ShapeDtypeStruct((1, SQ, HQ * DH), jnp.float32),
        in_specs=[pl.BlockSpec(memory_space=pltpu.VMEM)] * 5,
        out_specs=pl.BlockSpec(memory_space=pltpu.VMEM),
        scratch_shapes=[
            pltpu.VMEM((HQ, SQ, DH), jnp.float32),
            pltpu.VMEM((SQ, 2 * HQ), jnp.float32),
            pltpu.VMEM((HQ, SQ, DH), jnp.int8),
            pltpu.VMEM((HQ // 2, N_DEV - 1, 2, CHUNK, DH),
                       jnp.int8),
            pltpu.VMEM((N_DEV - 1, CHUNK, 2 * HQ), jnp.float32),
            pltpu.VMEM((CHUNK, HQ * DH), jnp.int8),
            pltpu.VMEM((N_DEV - 1, CHUNK, HQ * DH), jnp.int8),
            pltpu.VMEM((CHUNK,), jnp.float32),
            pltpu.VMEM((N_DEV - 1, CHUNK), jnp.float32),
            pltpu.SemaphoreType.DMA((HQ // 2, N_DEV - 1)),
            pltpu.SemaphoreType.DMA((HQ // 2, N_DEV - 1)),
            pltpu.SemaphoreType.DMA((N_DEV - 1,)),
            pltpu.SemaphoreType.DMA((N_DEV - 1,)),
            pltpu.SemaphoreType.DMA((N_DEV - 1,)),
            pltpu.SemaphoreType.DMA((N_DEV - 1,)),
            pltpu.SemaphoreType.DMA((N_DEV - 1,)),
            pltpu.SemaphoreType.DMA((N_DEV - 1,)),
        ],
        compiler_params=(None if (_SKIP_RS and _SKIP_AG)
                         else pltpu.CompilerParams(collective_id=0)),
    )(x, Wq, K_ext, V_ext, Wo)


# device time: 44617 ns/iter; 1.3223x vs baseline; 1.0131x over previous
import functools
import os

import jax
import jax.numpy as jnp
from jax import lax
from jax.experimental import pallas as pl
from jax.experimental.pallas import tpu as pltpu

N_DEV = 8
SQ = 1024
SKV = 1024
HQ = 8
DH = 128
CHUNK = SQ // N_DEV
NRES = 4
GRP = SQ // NRES
SCALE = 0.08838834764831843

_SKIP_RS = os.environ.get("K_SKIP_RS") == "1"
_SKIP_AG = os.environ.get("K_SKIP_AG") == "1"


def kernel(x, Wq, K_ext, V_ext, Wo):
    def body(x_ref, wq_ref, k_ref, v_ref, wo_ref, out_ref,
             acc_ref, lw_ref, acc_i8, rs_acc_slots, rs_lw_slots,
             ag_send, ag_slots, ag_sc_send, ag_sc_slots,
             acc_ssem, acc_rsem, l_ssem, l_rsem, ag_ssem, ag_rsem,
             sc_ssem, sc_rsem):
        my = lax.axis_index("i")

        if not (_SKIP_RS and _SKIP_AG):
            barrier = pltpu.get_barrier_semaphore()
            for o in range(1, N_DEV):
                peer = lax.rem(my + o, N_DEV)
                pl.semaphore_signal(barrier, inc=1, device_id=(peer,),
                                    device_id_type=pl.DeviceIdType.MESH)
            pl.semaphore_wait(barrier, N_DEV - 1)

        xb = x_ref[0].astype(jnp.bfloat16)
        wqb = wq_ref[...].astype(jnp.bfloat16)
        q = lax.dot_general(xb, wqb, (((1,), (0,)), ((), ())),
                            preferred_element_type=jnp.float32)
        q = (q * SCALE).astype(jnp.bfloat16)

        qp = q.reshape(NRES, NRES, 64, HQ * DH).transpose(1, 0, 2, 3)
        qp = qp.reshape(NRES, GRP, HQ * DH)
        kp = k_ref[0].astype(jnp.bfloat16).reshape(
            NRES, NRES, 64, HQ, DH).transpose(1, 0, 2, 3, 4)
        kp = kp.reshape(NRES, GRP, HQ, DH)
        vp = v_ref[0].astype(jnp.bfloat16).reshape(
            NRES, NRES, 64, HQ, DH).transpose(1, 0, 2, 3, 4)
        vp = vp.reshape(NRES, GRP, HQ, DH)

        def rs_descriptor(c, hp):
            j = lax.rem(c - my - 1 + N_DEV, N_DEV)
            return pltpu.make_async_remote_copy(
                src_ref=acc_i8.at[2 * hp:2 * hp + 2, pl.ds(c * CHUNK, CHUNK), :],
                dst_ref=rs_acc_slots.at[hp, j],
                send_sem=acc_ssem.at[hp, j], recv_sem=acc_rsem.at[hp, j],
                device_id=(c,), device_id_type=pl.DeviceIdType.MESH)

        def rs_lw_descriptor(c):
            j = lax.rem(c - my - 1 + N_DEV, N_DEV)
            return pltpu.make_async_remote_copy(
                src_ref=lw_ref.at[pl.ds(c * CHUNK, CHUNK)],
                dst_ref=rs_lw_slots.at[j],
                send_sem=l_ssem.at[j], recv_sem=l_rsem.at[j],
                device_id=(c,), device_id_type=pl.DeviceIdType.MESH)

        for h in range(HQ):
            for r in range(NRES):
                rrows = pl.ds(r * GRP, GRP)
                qrh = qp[r, :, h * DH:(h + 1) * DH]
                s = lax.dot_general(qrh, kp[r, :, h, :],
                                    (((1,), (1,)), ((), ())),
                                    preferred_element_type=jnp.float32)
                w = jnp.exp(s)
                lw_ref[rrows, h] = jnp.sum(w, axis=1)
                acc_ref[h, rrows, :] = lax.dot_general(
                    w.astype(jnp.bfloat16), vp[r, :, h, :],
                    (((1,), (0,)), ((), ())),
                    preferred_element_type=jnp.float32)
            acc_h = acc_ref[h]
            qsc = (jnp.max(jnp.abs(acc_h), axis=1, keepdims=True)
                   * (1.0 / 127.0) + 1e-20)
            acc_i8[h] = jnp.round(acc_h / qsc).astype(jnp.int8)
            lw_ref[:, 8 + h] = qsc[:, 0]
            if not _SKIP_RS and h % 2 == 1:
                for c in range(N_DEV):
                    @pl.when(my != c)
                    def _(c=c, hp=h // 2):
                        rs_descriptor(c, hp).start()

        if not _SKIP_RS:
            for c in range(N_DEV):
                @pl.when(my != c)
                def _(c=c):
                    rs_lw_descriptor(c).start()

        myl = pl.ds(my * CHUNK, CHUNK)
        if not _SKIP_RS:
            for hp in range(HQ // 2):
                for j in range(N_DEV - 1):
                    pltpu.make_async_remote_copy(
                        src_ref=rs_acc_slots.at[hp, j],
                        dst_ref=rs_acc_slots.at[hp, j],
                        send_sem=acc_ssem.at[hp, j], recv_sem=acc_rsem.at[hp, j],
                        device_id=(my,),
                        device_id_type=pl.DeviceIdType.MESH).wait_recv()
            for j in range(N_DEV - 1):
                pltpu.make_async_remote_copy(
                    src_ref=rs_lw_slots.at[j], dst_ref=rs_lw_slots.at[j],
                    send_sem=l_ssem.at[j], recv_sem=l_rsem.at[j],
                    device_id=(my,),
                    device_id_type=pl.DeviceIdType.MESH).wait_recv()

        tot_l = lw_ref[myl, 0:8]
        lw_in = []
        if not _SKIP_RS:
            lw_in = [rs_lw_slots[j] for j in range(N_DEV - 1)]
            for j in range(N_DEV - 1):
                tot_l = tot_l + lw_in[j][:, 0:8]

        def half_base(c, half):
            return (8 * lax.rem(c, 2) + lax.div(c, 2)) * 64 + 256 * half

        wob = wo_ref[...].astype(jnp.bfloat16)
        ag_rdmas = []
        for half in range(2):
            hsl = slice(64 * half, 64 * half + 64)
            mh = pl.ds(my * CHUNK + 64 * half, 64)
            ctx_parts = []
            for h in range(HQ):
                tot_h = acc_ref[h, mh, :]
                if not _SKIP_RS:
                    for j in range(N_DEV - 1):
                        tot_h = tot_h + (
                            rs_acc_slots[h // 2, j, h % 2][hsl].astype(
                                jnp.float32)
                            * lw_in[j][hsl, 8 + h, None])
                ctx_parts.append((tot_h / tot_l[hsl, h, None])
                                 .astype(jnp.bfloat16))
            ctxb = jnp.concatenate(ctx_parts, axis=1)
            out_half = lax.dot_general(ctxb, wob, (((1,), (0,)), ((), ())),
                                       preferred_element_type=jnp.float32)
            row_scale = (jnp.max(jnp.abs(out_half), axis=1, keepdims=True)
                         * (1.0 / 127.0) + 1e-20)
            ag_send[half] = jnp.round(out_half / row_scale).astype(jnp.int8)
            ag_sc_send[half] = row_scale[:, 0]
            out_ref[0, pl.ds(half_base(my, half), 64), :] = out_half

            if not _SKIP_AG:
                for o in range(1, N_DEV):
                    peer = lax.rem(my + o, N_DEV)
                    j = N_DEV - 1 - o
                    rdma = pltpu.make_async_remote_copy(
                        src_ref=ag_send.at[half], dst_ref=ag_slots.at[j, half],
                        send_sem=ag_ssem.at[half, j],
                        recv_sem=ag_rsem.at[half, j],
                        device_id=(peer,), device_id_type=pl.DeviceIdType.MESH)
                    rdma.start()
                    rdma_sc = pltpu.make_async_remote_copy(
                        src_ref=ag_sc_send.at[half],
                        dst_ref=ag_sc_slots.at[j, half],
                        send_sem=sc_ssem.at[half, j],
                        recv_sem=sc_rsem.at[half, j],
                        device_id=(peer,), device_id_type=pl.DeviceIdType.MESH)
                    rdma_sc.start()
                    ag_rdmas.extend((rdma, rdma_sc))

        if not _SKIP_AG:
            for half in range(2):
                for j in range(N_DEV - 1):
                    pltpu.make_async_remote_copy(
                        src_ref=ag_slots.at[j, half],
                        dst_ref=ag_slots.at[j, half],
                        send_sem=ag_ssem.at[half, j],
                        recv_sem=ag_rsem.at[half, j], device_id=(my,),
                        device_id_type=pl.DeviceIdType.MESH).wait_recv()
                    pltpu.make_async_remote_copy(
                        src_ref=ag_sc_slots.at[j, half],
                        dst_ref=ag_sc_slots.at[j, half],
                        send_sem=sc_ssem.at[half, j],
                        recv_sem=sc_rsem.at[half, j], device_id=(my,),
                        device_id_type=pl.DeviceIdType.MESH).wait_recv()
                    src = lax.rem(my + 1 + j, N_DEV)
                    out_ref[0, pl.ds(half_base(src, half), 64), :] = (
                        ag_slots[j, half].astype(jnp.float32)
                        * ag_sc_slots[j, half][:, None])

        if not _SKIP_RS:
            for c in range(N_DEV):
                @pl.when(my != c)
                def _(c=c):
                    for hp in range(HQ // 2):
                        rs_descriptor(c, hp).wait_send()
                    rs_lw_descriptor(c).wait_send()
        for rdma in ag_rdmas:
            rdma.wait_send()


    return pl.pallas_call(
        body,
        out_shape=jax.ShapeDtypeStruct((1, SQ, HQ * DH), jnp.float32),
        in_specs=[pl.BlockSpec(memory_space=pltpu.VMEM)] * 5,
        out_specs=pl.BlockSpec(memory_space=pltpu.VMEM),
        scratch_shapes=[
            pltpu.VMEM((HQ, SQ, DH), jnp.float32),
            pltpu.VMEM((SQ, 2 * HQ), jnp.float32),
            pltpu.VMEM((HQ, SQ, DH), jnp.int8),
            pltpu.VMEM((HQ // 2, N_DEV - 1, 2, CHUNK, DH),
                       jnp.int8),
            pltpu.VMEM((N_DEV - 1, CHUNK, 2 * HQ), jnp.float32),
            pltpu.VMEM((2, CHUNK // 2, HQ * DH), jnp.int8),
            pltpu.VMEM((N_DEV - 1, 2, CHUNK // 2, HQ * DH),
                       jnp.int8),
            pltpu.VMEM((2, CHUNK // 2), jnp.float32),
            pltpu.VMEM((N_DEV - 1, 2, CHUNK // 2), jnp.float32),
            pltpu.SemaphoreType.DMA((HQ // 2, N_DEV - 1)),
            pltpu.SemaphoreType.DMA((HQ // 2, N_DEV - 1)),
            pltpu.SemaphoreType.DMA((N_DEV - 1,)),
            pltpu.SemaphoreType.DMA((N_DEV - 1,)),
            pltpu.SemaphoreType.DMA((2, N_DEV - 1)),
            pltpu.SemaphoreType.DMA((2, N_DEV - 1)),
            pltpu.SemaphoreType.DMA((2, N_DEV - 1)),
            pltpu.SemaphoreType.DMA((2, N_DEV - 1)),
        ],
        compiler_params=(None if (_SKIP_RS and _SKIP_AG)
                         else pltpu.CompilerParams(collective_id=0)),
    )(x, Wq, K_ext, V_ext, Wo)
